# Initial kernel scaffold; baseline (speedup 1.0000x reference)
import jax
import jax.numpy as jnp
from jax.experimental import pallas as pl


def kernel(x, edge_index, W1, b1, W2, b2):
    raise NotImplementedError("write your pallas kernel here")


# speedup vs baseline: 12.7535x; 12.7535x over previous
import functools

import jax
import jax.numpy as jnp
from jax import lax
from jax.experimental import pallas as pl
from jax.experimental.pallas import tpu as pltpu
from jax.experimental.pallas import tpu_sc as plsc

_N = 10000
_E = 320000
_D = 128
_NC = 2
_NS = 16
_NW = _NC * _NS
_EPT = _E // _NW
_KB = 80
_NB = _EPT // _KB
_NP = 10240
_HR = _NP // 128
_NH = 5120
_NPH = 5376
_RPH = _NPH // _NS
_DPT = _NH // _NS
_LANE = 16

_MESH = plsc.VectorSubcoreMesh(core_axis_name="c", subcore_axis_name="s")


def _sc_aggregate_body(h_hbm, src_hbm, d1_hbm, d2_hbm, zrows_hbm, out_hbm,
                       sidx, didx1, didx2, buf0, buf1, acc, sem0, sem1):
    c = lax.axis_index("c")
    s = lax.axis_index("s")
    wid = c * _NS + s
    pltpu.sync_copy(src_hbm.at[wid], sidx)
    pltpu.sync_copy(d1_hbm.at[wid], didx1)
    pltpu.sync_copy(d2_hbm.at[wid], didx2)

    for p, didxp in ((0, didx1), (1, didx2)):
        pltpu.sync_copy(zrows_hbm, acc.at[pl.ds(s * _RPH, _RPH)])
        plsc.subcore_barrier()

        def body(i, carry):
            j = i * 2
            cp0 = pltpu.async_copy(h_hbm.at[sidx.at[j]], buf0, sem0)
            cp1 = pltpu.async_copy(h_hbm.at[sidx.at[j + 1]], buf1, sem1)
            cp0.wait()
            pltpu.sync_copy(buf0, acc.at[didxp.at[j]], add=True)
            cp1.wait()
            pltpu.sync_copy(buf1, acc.at[didxp.at[j + 1]], add=True)
            return carry

        lax.fori_loop(0, _NB // 2, body, 0, unroll=False)
        cp = pltpu.async_copy(h_hbm.at[sidx.at[_NB - 1]], buf0, sem0)
        cp.wait()
        pltpu.sync_copy(buf0, acc.at[didxp.at[_NB - 1]], add=True)

        plsc.subcore_barrier()
        pltpu.sync_copy(acc.at[pl.ds(s * _DPT, _DPT)],
                        out_hbm.at[c, pl.ds(p * _NH + s * _DPT, _DPT)])
        plsc.subcore_barrier()


_sc_aggregate = functools.partial(
    pl.kernel,
    out_type=jax.ShapeDtypeStruct((_NC, _NP, _D), jnp.float32),
    mesh=_MESH,
    scratch_types=[
        pltpu.VMEM((_NB, _KB), jnp.int32),
        pltpu.VMEM((_NB, _KB), jnp.int32),
        pltpu.VMEM((_NB, _KB), jnp.int32),
        pltpu.VMEM((_KB, _D), jnp.float32),
        pltpu.VMEM((_KB, _D), jnp.float32),
        pltpu.VMEM_SHARED((_NPH, _D), jnp.float32),
        pltpu.SemaphoreType.DMA,
        pltpu.SemaphoreType.DMA,
    ],
    compiler_params=pltpu.CompilerParams(needs_layout_passes=False),
)(_sc_aggregate_body)


def _sc_degree_body(dst_hbm, zh_hbm, out_hbm, didx, hist, idv, acc):
    c = lax.axis_index("c")
    s = lax.axis_index("s")
    wid = c * _NS + s
    pltpu.sync_copy(dst_hbm.at[wid], didx)
    pltpu.sync_copy(zh_hbm, hist)

    @pl.when(s == 0)
    def _():
        pltpu.sync_copy(zh_hbm, acc)

    def mkid(k, carry):
        idv[0, pl.ds(k * _LANE, _LANE)] = lax.iota(jnp.int32, _LANE) + k * _LANE
        return carry

    lax.fori_loop(0, _HR // _LANE, mkid, 0, unroll=False)

    ones = jnp.ones((_LANE,), jnp.float32)

    def body(i, carry):
        for k in range(_KB // _LANE):
            d = didx[i, pl.ds(k * _LANE, _LANE)]
            plsc.addupdate_scatter(hist, [d >> 7, d & 127], ones)
        return carry

    lax.fori_loop(0, _NB, body, 0, unroll=False)

    plsc.subcore_barrier()
    pltpu.sync_copy(hist, acc.at[idv.at[0]], add=True)
    plsc.subcore_barrier()

    @pl.when(s < _HR // 8)
    def _():
        pltpu.sync_copy(acc.at[pl.ds(s * 8, 8)],
                        out_hbm.at[c, pl.ds(s * 8, 8)])


_sc_degree = functools.partial(
    pl.kernel,
    out_type=jax.ShapeDtypeStruct((_NC, _HR, _D), jnp.float32),
    mesh=_MESH,
    scratch_types=[
        pltpu.VMEM((_NB, _KB), jnp.int32),
        pltpu.VMEM((_HR, _D), jnp.float32),
        pltpu.VMEM((1, _HR), jnp.int32),
        pltpu.VMEM_SHARED((_HR, _D), jnp.float32),
    ],
    compiler_params=pltpu.CompilerParams(needs_layout_passes=False),
)(_sc_degree_body)


_BM = 1000


def _r_from_deg(dega_ref, degb_ref):
    deg = dega_ref[...] + degb_ref[...] + 1.0
    return lax.rsqrt(deg)


def _tc_scale_matmul_body(dega_ref, degb_ref, x_ref, w_ref, o_ref):
    r = _r_from_deg(dega_ref, degb_ref)
    o_ref[...] = jnp.dot(x_ref[...], w_ref[...],
                         preferred_element_type=jnp.float32,
                         precision=lax.Precision.HIGHEST) * r


def _tc_layer_body(dega_ref, degb_ref, agga_ref, aggb_ref, hp_ref, b_ref,
                   w_ref, o_ref):
    r = _r_from_deg(dega_ref, degb_ref)
    z = jnp.maximum(r * (agga_ref[...] + aggb_ref[...] + hp_ref[...])
                    + b_ref[...], 0.0)
    o_ref[...] = jnp.dot(z, w_ref[...], preferred_element_type=jnp.float32,
                         precision=lax.Precision.HIGHEST) * r


def _tc_final_body(dega_ref, degb_ref, agga_ref, aggb_ref, hp_ref, b_ref,
                   o_ref):
    r = _r_from_deg(dega_ref, degb_ref)
    o_ref[...] = jnp.maximum(r * (agga_ref[...] + aggb_ref[...] + hp_ref[...])
                             + b_ref[...], 0.0)


def _tc_prep_idx_body(dst_ref, o1_ref, o2_ref):
    d = dst_ref[...]
    g = _NH + (d & 255)
    in1 = d < _NH
    o1_ref[...] = jnp.where(in1, d, g)
    o2_ref[...] = jnp.where(in1, g, d - _NH)


_idx_spec = pl.BlockSpec((_E // _D, _D), lambda: (0, 0))

_tc_prep_idx = pl.pallas_call(
    _tc_prep_idx_body,
    in_specs=[_idx_spec],
    out_specs=[_idx_spec, _idx_spec],
    out_shape=[jax.ShapeDtypeStruct((_E // _D, _D), jnp.int32),
               jax.ShapeDtypeStruct((_E // _D, _D), jnp.int32)],
)


_deg_spec = pl.BlockSpec((_BM, 1), lambda i: (i, 0))
_row_spec = pl.BlockSpec((_BM, _D), lambda i: (i, 0))
_w_spec = pl.BlockSpec((_D, _D), lambda i: (0, 0))
_b_spec = pl.BlockSpec((1, _D), lambda i: (0, 0))

_tc_scale_matmul = pl.pallas_call(
    _tc_scale_matmul_body,
    grid=(_N // _BM,),
    in_specs=[_deg_spec, _deg_spec, _row_spec, _w_spec],
    out_specs=_row_spec,
    out_shape=jax.ShapeDtypeStruct((_N, _D), jnp.float32),
)

_tc_layer = pl.pallas_call(
    _tc_layer_body,
    grid=(_N // _BM,),
    in_specs=[_deg_spec, _deg_spec, _row_spec, _row_spec, _row_spec, _b_spec,
              _w_spec],
    out_specs=_row_spec,
    out_shape=jax.ShapeDtypeStruct((_N, _D), jnp.float32),
)

_tc_final = pl.pallas_call(
    _tc_final_body,
    grid=(_N // _BM,),
    in_specs=[_deg_spec, _deg_spec, _row_spec, _row_spec, _row_spec, _b_spec],
    out_specs=_row_spec,
    out_shape=jax.ShapeDtypeStruct((_N, _D), jnp.float32),
)


def kernel(x, edge_index, W1, b1, W2, b2):
    src = edge_index[0].reshape(_NW, _NB, _KB)
    dst = edge_index[1].reshape(_NW, _NB, _KB)
    zrows = jnp.zeros((_RPH, _D), jnp.float32)
    zh = jnp.zeros((_HR, _D), jnp.float32)
    b1r = b1.reshape(1, _D)
    b2r = b2.reshape(1, _D)

    d1f, d2f = _tc_prep_idx(edge_index[1].reshape(_E // _D, _D))
    d1 = d1f.reshape(_NW, _NB, _KB)
    d2 = d2f.reshape(_NW, _NB, _KB)

    dego = _sc_degree(dst, zh)
    dega = dego[0].reshape(_NP)[:_N, None]
    degb = dego[1].reshape(_NP)[:_N, None]

    h1p = _tc_scale_matmul(dega, degb, x, W1)
    a1 = _sc_aggregate(h1p, src, d1, d2, zrows)
    h2p = _tc_layer(dega, degb, a1[0, :_N], a1[1, :_N], h1p, b1r, W2)
    a2 = _sc_aggregate(h2p, src, d1, d2, zrows)
    out = _tc_final(dega, degb, a2[0, :_N], a2[1, :_N], h2p, b2r)
    return out


# speedup vs baseline: 14.3560x; 1.1256x over previous
import functools

import jax
import jax.numpy as jnp
from jax import lax
from jax.experimental import pallas as pl
from jax.experimental.pallas import tpu as pltpu
from jax.experimental.pallas import tpu_sc as plsc

_N = 10000
_E = 320000
_D = 128
_NC = 2
_NS = 16
_NW = _NC * _NS
_EPT = _E // _NW
_KB = 80
_NB = _EPT // _KB
_NP = 10240
_HR = _NP // 128
_NH = 5120
_NPH = 5376
_RPH = _NPH // _NS
_DPT = _NH // _NS
_LANE = 16

_MESH = plsc.VectorSubcoreMesh(core_axis_name="c", subcore_axis_name="s")


def _sc_partition_body(src_hbm, dst_hbm, pads_hbm, padd_hbm,
                       ps1_hbm, pd1_hbm, ps2_hbm, pd2_hbm, cnt_hbm,
                       sidx, didx, ps1, pd1, ps2, pd2, cntv):
    c = lax.axis_index("c")
    s = lax.axis_index("s")
    wid = c * _NS + s
    pltpu.sync_copy(src_hbm.at[wid], sidx)
    pltpu.sync_copy(dst_hbm.at[wid], didx)
    pltpu.sync_copy(pads_hbm, ps1)
    pltpu.sync_copy(pads_hbm, ps2)
    pltpu.sync_copy(padd_hbm, pd1)
    pltpu.sync_copy(padd_hbm, pd2)

    zero16 = jnp.zeros((_LANE,), jnp.int32)

    def chunk(i, carry):
        off1, off2 = carry
        for k in range(_KB // _LANE):
            d = didx[i, pl.ds(k * _LANE, _LANE)]
            sv = sidx[i, pl.ds(k * _LANE, _LANE)]
            m1 = d < _NH
            m2 = jnp.logical_not(m1)
            pos1 = off1 + plsc.cumsum(m1.astype(jnp.int32)) - 1
            pos2 = off2 + plsc.cumsum(m2.astype(jnp.int32)) - 1
            plsc.store_scatter(ps1, [pos1], sv, mask=m1)
            plsc.store_scatter(pd1, [pos1], d, mask=m1)
            plsc.store_scatter(ps2, [pos2], sv, mask=m2)
            plsc.store_scatter(pd2, [pos2], d - _NH, mask=m2)
            off1 = off1 + plsc.all_reduce_population_count(m1)
            off2 = off2 + plsc.all_reduce_population_count(m2)
        return off1, off2

    off1, off2 = lax.fori_loop(0, _NB, chunk, (zero16, zero16), unroll=False)
    cntv[0, pl.ds(0, _LANE)] = off1
    cntv[0, pl.ds(_LANE, _LANE)] = off2
    for t in range(2, 8):
        cntv[0, pl.ds(t * _LANE, _LANE)] = zero16

    pltpu.sync_copy(ps1, ps1_hbm.at[wid])
    pltpu.sync_copy(pd1, pd1_hbm.at[wid])
    pltpu.sync_copy(ps2, ps2_hbm.at[wid])
    pltpu.sync_copy(pd2, pd2_hbm.at[wid])
    pltpu.sync_copy(cntv, cnt_hbm.at[wid])


_flat_idx = jax.ShapeDtypeStruct((_NW, _EPT), jnp.int32)

_sc_partition = functools.partial(
    pl.kernel,
    out_type=[_flat_idx, _flat_idx, _flat_idx, _flat_idx,
              jax.ShapeDtypeStruct((_NW, 1, 128), jnp.int32)],
    mesh=_MESH,
    scratch_types=[
        pltpu.VMEM((_NB, _KB), jnp.int32),
        pltpu.VMEM((_NB, _KB), jnp.int32),
        pltpu.VMEM((_EPT,), jnp.int32),
        pltpu.VMEM((_EPT,), jnp.int32),
        pltpu.VMEM((_EPT,), jnp.int32),
        pltpu.VMEM((_EPT,), jnp.int32),
        pltpu.VMEM((1, 128), jnp.int32),
    ],
    compiler_params=pltpu.CompilerParams(needs_layout_passes=False),
)(_sc_partition_body)


def _sc_aggregate_body(h_hbm, ps1_hbm, pd1_hbm, ps2_hbm, pd2_hbm, cnt_hbm,
                       zrows_hbm, out_hbm,
                       sidx, didx, cntv, buf0, buf1, acc, sem0, sem1):
    c = lax.axis_index("c")
    s = lax.axis_index("s")
    wid = c * _NS + s
    pltpu.sync_copy(cnt_hbm.at[wid], cntv)

    for p, (ps_hbm, pd_hbm) in enumerate(((ps1_hbm, pd1_hbm),
                                          (ps2_hbm, pd2_hbm))):
        pltpu.sync_copy(ps_hbm.at[wid], sidx)
        pltpu.sync_copy(pd_hbm.at[wid], didx)
        pltpu.sync_copy(zrows_hbm, acc.at[pl.ds(s * _RPH, _RPH)])
        plsc.subcore_barrier()

        cntp = jnp.max(cntv[0, pl.ds(p * _LANE, _LANE)])
        nb = (cntp + (_KB - 1)) // _KB

        def body(i, carry):
            j = i * 2
            cp0 = pltpu.async_copy(h_hbm.at[sidx.at[j]], buf0, sem0)
            cp1 = pltpu.async_copy(h_hbm.at[sidx.at[j + 1]], buf1, sem1)
            cp0.wait()
            pltpu.sync_copy(buf0, acc.at[didx.at[j]], add=True)
            cp1.wait()
            pltpu.sync_copy(buf1, acc.at[didx.at[j + 1]], add=True)
            return carry

        lax.fori_loop(0, nb // 2, body, 0, unroll=False)

        @pl.when(nb % 2 == 1)
        def _():
            cp = pltpu.async_copy(h_hbm.at[sidx.at[nb - 1]], buf0, sem0)
            cp.wait()
            pltpu.sync_copy(buf0, acc.at[didx.at[nb - 1]], add=True)

        plsc.subcore_barrier()
        pltpu.sync_copy(acc.at[pl.ds(s * _DPT, _DPT)],
                        out_hbm.at[c, pl.ds(p * _NH + s * _DPT, _DPT)])
        plsc.subcore_barrier()


_sc_aggregate = functools.partial(
    pl.kernel,
    out_type=jax.ShapeDtypeStruct((_NC, _NP, _D), jnp.float32),
    mesh=_MESH,
    scratch_types=[
        pltpu.VMEM((_NB, _KB), jnp.int32),
        pltpu.VMEM((_NB, _KB), jnp.int32),
        pltpu.VMEM((1, 128), jnp.int32),
        pltpu.VMEM((_KB, _D), jnp.float32),
        pltpu.VMEM((_KB, _D), jnp.float32),
        pltpu.VMEM_SHARED((_NPH, _D), jnp.float32),
        pltpu.SemaphoreType.DMA,
        pltpu.SemaphoreType.DMA,
    ],
    compiler_params=pltpu.CompilerParams(needs_layout_passes=False),
)(_sc_aggregate_body)


def _sc_degree_body(dst_hbm, zh_hbm, out_hbm, didx, hist, idv, acc):
    c = lax.axis_index("c")
    s = lax.axis_index("s")
    wid = c * _NS + s
    pltpu.sync_copy(dst_hbm.at[wid], didx)
    pltpu.sync_copy(zh_hbm, hist)

    @pl.when(s == 0)
    def _():
        pltpu.sync_copy(zh_hbm, acc)

    def mkid(k, carry):
        idv[0, pl.ds(k * _LANE, _LANE)] = lax.iota(jnp.int32, _LANE) + k * _LANE
        return carry

    lax.fori_loop(0, _HR // _LANE, mkid, 0, unroll=False)

    ones = jnp.ones((_LANE,), jnp.float32)

    def body(i, carry):
        for k in range(_KB // _LANE):
            d = didx[i, pl.ds(k * _LANE, _LANE)]
            plsc.addupdate_scatter(hist, [d >> 7, d & 127], ones)
        return carry

    lax.fori_loop(0, _NB, body, 0, unroll=False)

    plsc.subcore_barrier()
    pltpu.sync_copy(hist, acc.at[idv.at[0]], add=True)
    plsc.subcore_barrier()

    @pl.when(s < _HR // 8)
    def _():
        pltpu.sync_copy(acc.at[pl.ds(s * 8, 8)],
                        out_hbm.at[c, pl.ds(s * 8, 8)])


_sc_degree = functools.partial(
    pl.kernel,
    out_type=jax.ShapeDtypeStruct((_NC, _HR, _D), jnp.float32),
    mesh=_MESH,
    scratch_types=[
        pltpu.VMEM((_NB, _KB), jnp.int32),
        pltpu.VMEM((_HR, _D), jnp.float32),
        pltpu.VMEM((1, _HR), jnp.int32),
        pltpu.VMEM_SHARED((_HR, _D), jnp.float32),
    ],
    compiler_params=pltpu.CompilerParams(needs_layout_passes=False),
)(_sc_degree_body)


_BM = 1000


def _r_from_deg(dega_ref, degb_ref):
    deg = dega_ref[...] + degb_ref[...] + 1.0
    return lax.rsqrt(deg)


def _tc_scale_matmul_body(dega_ref, degb_ref, x_ref, w_ref, o_ref):
    r = _r_from_deg(dega_ref, degb_ref)
    o_ref[...] = jnp.dot(x_ref[...], w_ref[...],
                         preferred_element_type=jnp.float32,
                         precision=lax.Precision.HIGHEST) * r


def _tc_layer_body(dega_ref, degb_ref, agga_ref, aggb_ref, hp_ref, b_ref,
                   w_ref, o_ref):
    r = _r_from_deg(dega_ref, degb_ref)
    z = jnp.maximum(r * (agga_ref[...] + aggb_ref[...] + hp_ref[...])
                    + b_ref[...], 0.0)
    o_ref[...] = jnp.dot(z, w_ref[...], preferred_element_type=jnp.float32,
                         precision=lax.Precision.HIGHEST) * r


def _tc_final_body(dega_ref, degb_ref, agga_ref, aggb_ref, hp_ref, b_ref,
                   o_ref):
    r = _r_from_deg(dega_ref, degb_ref)
    o_ref[...] = jnp.maximum(r * (agga_ref[...] + aggb_ref[...] + hp_ref[...])
                             + b_ref[...], 0.0)


_deg_spec = pl.BlockSpec((_BM, 1), lambda i: (i, 0))
_row_spec = pl.BlockSpec((_BM, _D), lambda i: (i, 0))
_w_spec = pl.BlockSpec((_D, _D), lambda i: (0, 0))
_b_spec = pl.BlockSpec((1, _D), lambda i: (0, 0))

_tc_scale_matmul = pl.pallas_call(
    _tc_scale_matmul_body,
    grid=(_N // _BM,),
    in_specs=[_deg_spec, _deg_spec, _row_spec, _w_spec],
    out_specs=_row_spec,
    out_shape=jax.ShapeDtypeStruct((_N, _D), jnp.float32),
)

_tc_layer = pl.pallas_call(
    _tc_layer_body,
    grid=(_N // _BM,),
    in_specs=[_deg_spec, _deg_spec, _row_spec, _row_spec, _row_spec, _b_spec,
              _w_spec],
    out_specs=_row_spec,
    out_shape=jax.ShapeDtypeStruct((_N, _D), jnp.float32),
)

_tc_final = pl.pallas_call(
    _tc_final_body,
    grid=(_N // _BM,),
    in_specs=[_deg_spec, _deg_spec, _row_spec, _row_spec, _row_spec, _b_spec],
    out_specs=_row_spec,
    out_shape=jax.ShapeDtypeStruct((_N, _D), jnp.float32),
)


def kernel(x, edge_index, W1, b1, W2, b2):
    src = edge_index[0].reshape(_NW, _NB, _KB)
    dst = edge_index[1].reshape(_NW, _NB, _KB)
    zrows = jnp.zeros((_RPH, _D), jnp.float32)
    zh = jnp.zeros((_HR, _D), jnp.float32)
    b1r = b1.reshape(1, _D)
    b2r = b2.reshape(1, _D)

    pads = jnp.zeros((_EPT,), jnp.int32)
    padd = (_NH + (jnp.arange(_EPT, dtype=jnp.int32) % 256)).astype(jnp.int32)
    ps1f, pd1f, ps2f, pd2f, cnt = _sc_partition(src, dst, pads, padd)
    ps1 = ps1f.reshape(_NW, _NB, _KB)
    pd1 = pd1f.reshape(_NW, _NB, _KB)
    ps2 = ps2f.reshape(_NW, _NB, _KB)
    pd2 = pd2f.reshape(_NW, _NB, _KB)

    dego = _sc_degree(dst, zh)
    dega = dego[0].reshape(_NP)[:_N, None]
    degb = dego[1].reshape(_NP)[:_N, None]

    h1p = _tc_scale_matmul(dega, degb, x, W1)
    a1 = _sc_aggregate(h1p, ps1, pd1, ps2, pd2, cnt, zrows)
    h2p = _tc_layer(dega, degb, a1[0, :_N], a1[1, :_N], h1p, b1r, W2)
    a2 = _sc_aggregate(h2p, ps1, pd1, ps2, pd2, cnt, zrows)
    out = _tc_final(dega, degb, a2[0, :_N], a2[1, :_N], h2p, b2r)
    return out
